# Initial kernel scaffold; baseline (speedup 1.0000x reference)
import jax
import jax.numpy as jnp
import numpy as np
from jax.experimental import pallas as pl

STRIDES = (8, 16, 32, 64, 128)
FEATURE_SHAPES = ((128, 128), (64, 64), (32, 32), (16, 16), (8, 8))
POS_SCALE = 0.2
IGNORE_SCALE = 0.5
NUM_CLASSES = 80
TILE = 64
NUM_LOC = sum(h * w for h, w in FEATURE_SHAPES)
NUM_TILES = NUM_LOC // TILE


def _build_meta():
    cols = []
    for lid, (stride, (fh, fw)) in enumerate(zip(STRIDES, FEATURE_SHAPES)):
        ys, xs = np.meshgrid(np.arange(fh), np.arange(fw), indexing="ij")
        xs = xs.reshape(-1).astype(np.float32)
        ys = ys.reshape(-1).astype(np.float32)
        sx = (xs + 0.5) * stride
        sy = (ys + 0.5) * stride
        n = fh * fw
        cols.append(
            np.stack(
                [xs, ys, sx, sy,
                 np.full(n, lid, np.float32),
                 np.full(n, fw, np.float32),
                 np.full(n, fh, np.float32),
                 np.full(n, 1.0 / stride, np.float32)],
                axis=1,
            )
        )
    meta = np.concatenate(cols, axis=0)
    return meta.reshape(NUM_TILES, TILE, 8)


_META = _build_meta()


def _fsaf_tile(box_ref, meta_ref, cls_ref, clsm_ref, npos_ref, regr_ref,
               regrm_ref):
    t = pl.program_id(1)
    rows = box_ref[0]
    x1 = rows[0:1, :]
    y1 = rows[1:2, :]
    x2 = rows[2:3, :]
    y2 = rows[3:4, :]
    lab = rows[4:5, :]
    blv = rows[5:6, :]

    meta = meta_ref[0]
    xc = meta[:, 0:1]
    yc = meta[:, 1:2]
    sx = meta[:, 2:3]
    sy = meta[:, 3:4]
    lvl = meta[:, 4:5]
    fw = meta[:, 5:6]
    fh = meta[:, 6:7]
    inv_s = meta[:, 7:8]

    px1 = x1 * inv_s
    py1 = y1 * inv_s
    px2 = x2 * inv_s
    py2 = y2 * inv_s
    cx = (px1 + px2) * 0.5
    cy = (py1 + py2) * 0.5
    dw = px2 - px1
    dh = py2 - py1

    def bounds(scale):
        hw = dw * scale * 0.5
        hh = dh * scale * 0.5
        bx1 = jnp.clip(jnp.floor(cx - hw), 0.0, fw - 1.0)
        by1 = jnp.clip(jnp.floor(cy - hh), 0.0, fh - 1.0)
        bx2 = jnp.clip(jnp.ceil(cx + hw), 1.0, fw)
        by2 = jnp.clip(jnp.ceil(cy + hh), 1.0, fh)
        bx2 = jnp.maximum(bx2, bx1 + 1.0)
        by2 = jnp.maximum(by2, by1 + 1.0)
        return bx1, by1, bx2, by2

    at_level = blv == lvl
    qx1, qy1, qx2, qy2 = bounds(POS_SCALE)
    ix1, iy1, ix2, iy2 = bounds(IGNORE_SCALE)
    in_pos = (at_level & (xc >= qx1) & (xc < qx2)
              & (yc >= qy1) & (yc < qy2))
    in_ign = (at_level & (xc >= ix1) & (xc < ix2)
              & (yc >= iy1) & (yc < iy2))

    area = (x2 - x1) * (y2 - y1)
    area_m = jnp.where(in_pos, area, 1e7)
    best = jnp.min(area_m, axis=1, keepdims=True)
    lane = jax.lax.broadcasted_iota(jnp.float32, area_m.shape, 1)
    sel = jnp.min(jnp.where(area_m == best, lane, 1e9), axis=1, keepdims=True)
    onehot = (lane == sel).astype(jnp.float32)
    sel_pos = (best < 1e7).astype(jnp.float32)

    sx1 = jnp.sum(onehot * x1, axis=1, keepdims=True)
    sy1 = jnp.sum(onehot * y1, axis=1, keepdims=True)
    sx2 = jnp.sum(onehot * x2, axis=1, keepdims=True)
    sy2 = jnp.sum(onehot * y2, axis=1, keepdims=True)
    slab = jnp.sum(onehot * lab, axis=1, keepdims=True)

    lch = (sx - sx1) / 4.0 * sel_pos
    tch = (sy - sy1) / 4.0 * sel_pos
    rch = (sx2 - sx) / 4.0 * sel_pos
    bch = (sy2 - sy) / 4.0 * sel_pos
    regr_ref[0] = jnp.concatenate([lch, tch, rch, bch], axis=1)

    cls_iota = jax.lax.broadcasted_iota(jnp.float32, (TILE, NUM_CLASSES), 1)
    cls_ref[0] = (cls_iota == slab).astype(jnp.float32) * sel_pos

    any_ign = jnp.max(in_ign.astype(jnp.float32), axis=1, keepdims=True)
    clsm_ref[0] = jnp.maximum(sel_pos, 1.0 - any_ign)
    regrm_ref[0] = sel_pos

    part = jnp.sum(sel_pos).reshape(1, 1, 1)

    @pl.when(t == 0)
    def _init():
        npos_ref[...] = part

    @pl.when(t != 0)
    def _acc():
        npos_ref[...] += part


def kernel(gt_box_levels, gt_boxes, feature_shapes):
    del feature_shapes
    batch, nb = gt_box_levels.shape
    rows = jnp.concatenate(
        [jnp.transpose(gt_boxes, (0, 2, 1)),
         gt_box_levels[:, None, :].astype(jnp.float32)],
        axis=1,
    )
    meta = jnp.asarray(_META)

    cls_t, cls_m, num_pos, regr_t, regr_m = pl.pallas_call(
        _fsaf_tile,
        grid=(batch, NUM_TILES),
        in_specs=[
            pl.BlockSpec((1, 6, nb), lambda b, t: (b, 0, 0)),
            pl.BlockSpec((1, TILE, 8), lambda b, t: (t, 0, 0)),
        ],
        out_specs=[
            pl.BlockSpec((1, TILE, NUM_CLASSES), lambda b, t: (b, t, 0)),
            pl.BlockSpec((1, TILE, 1), lambda b, t: (b, t, 0)),
            pl.BlockSpec((1, 1, 1), lambda b, t: (b, 0, 0)),
            pl.BlockSpec((1, TILE, 4), lambda b, t: (b, t, 0)),
            pl.BlockSpec((1, TILE, 1), lambda b, t: (b, t, 0)),
        ],
        out_shape=[
            jax.ShapeDtypeStruct((batch, NUM_LOC, NUM_CLASSES), jnp.float32),
            jax.ShapeDtypeStruct((batch, NUM_LOC, 1), jnp.float32),
            jax.ShapeDtypeStruct((batch, 1, 1), jnp.float32),
            jax.ShapeDtypeStruct((batch, NUM_LOC, 4), jnp.float32),
            jax.ShapeDtypeStruct((batch, NUM_LOC, 1), jnp.float32),
        ],
    )(rows, meta)

    return (cls_t,
            cls_m[..., 0] != 0.0,
            num_pos[:, 0, 0],
            regr_t,
            regr_m[..., 0] != 0.0)


# speedup vs baseline: 2.0375x; 2.0375x over previous
import jax
import jax.numpy as jnp
import numpy as np
from jax.experimental import pallas as pl

STRIDES = (8, 16, 32, 64, 128)
FEATURE_SHAPES = ((128, 128), (64, 64), (32, 32), (16, 16), (8, 8))
POS_SCALE = 0.2
IGNORE_SCALE = 0.5
NUM_CLASSES = 80
TILE = 64
NUM_LOC = sum(h * w for h, w in FEATURE_SHAPES)
NUM_TILES = NUM_LOC // TILE


def _build_meta():
    cols = []
    for lid, (stride, (fh, fw)) in enumerate(zip(STRIDES, FEATURE_SHAPES)):
        ys, xs = np.meshgrid(np.arange(fh), np.arange(fw), indexing="ij")
        xs = xs.reshape(-1).astype(np.float32)
        ys = ys.reshape(-1).astype(np.float32)
        sx = (xs + 0.5) * stride
        sy = (ys + 0.5) * stride
        n = fh * fw
        cols.append(
            np.stack(
                [xs, ys, sx, sy,
                 np.full(n, lid, np.float32),
                 np.full(n, fw, np.float32),
                 np.full(n, fh, np.float32),
                 np.full(n, 1.0 / stride, np.float32)],
                axis=1,
            )
        )
    meta = np.concatenate(cols, axis=0)
    return meta.reshape(NUM_TILES, TILE, 8)


_META = _build_meta()


def _fsaf_tile(box_ref, meta_ref, cls_ref, clsm_ref, npos_ref, regr_ref,
               regrm_ref):
    t = pl.program_id(1)
    rows = box_ref[0]
    x1 = rows[0:1, :]
    y1 = rows[1:2, :]
    x2 = rows[2:3, :]
    y2 = rows[3:4, :]
    lab = rows[4:5, :]
    blv = rows[5:6, :]

    meta = meta_ref[0]
    xc = meta[:, 0:1]
    yc = meta[:, 1:2]
    sx = meta[:, 2:3]
    sy = meta[:, 3:4]
    lvl = meta[:, 4:5]
    fw = meta[:, 5:6]
    fh = meta[:, 6:7]
    inv_s = meta[:, 7:8]

    px1 = x1 * inv_s
    py1 = y1 * inv_s
    px2 = x2 * inv_s
    py2 = y2 * inv_s
    cx = (px1 + px2) * 0.5
    cy = (py1 + py2) * 0.5
    dw = px2 - px1
    dh = py2 - py1

    def bounds(scale):
        hw = dw * scale * 0.5
        hh = dh * scale * 0.5
        bx1 = jnp.clip(jnp.floor(cx - hw), 0.0, fw - 1.0)
        by1 = jnp.clip(jnp.floor(cy - hh), 0.0, fh - 1.0)
        bx2 = jnp.clip(jnp.ceil(cx + hw), 1.0, fw)
        by2 = jnp.clip(jnp.ceil(cy + hh), 1.0, fh)
        bx2 = jnp.maximum(bx2, bx1 + 1.0)
        by2 = jnp.maximum(by2, by1 + 1.0)
        return bx1, by1, bx2, by2

    at_level = blv == lvl
    qx1, qy1, qx2, qy2 = bounds(POS_SCALE)
    ix1, iy1, ix2, iy2 = bounds(IGNORE_SCALE)
    in_pos = (at_level & (xc >= qx1) & (xc < qx2)
              & (yc >= qy1) & (yc < qy2))
    in_ign = (at_level & (xc >= ix1) & (xc < ix2)
              & (yc >= iy1) & (yc < iy2))

    area = (x2 - x1) * (y2 - y1)
    area_m = jnp.where(in_pos, area, 1e7)
    best = jnp.min(area_m, axis=1, keepdims=True)
    lane = jax.lax.broadcasted_iota(jnp.int32, area_m.shape, 1)
    sel = jnp.min(jnp.where(area_m == best, lane, 2**30), axis=1,
                  keepdims=True)
    onehot = (lane == sel).astype(jnp.float32)
    sel_pos = (best < 1e7).astype(jnp.float32)

    sx1 = jnp.sum(onehot * x1, axis=1, keepdims=True)
    sy1 = jnp.sum(onehot * y1, axis=1, keepdims=True)
    sx2 = jnp.sum(onehot * x2, axis=1, keepdims=True)
    sy2 = jnp.sum(onehot * y2, axis=1, keepdims=True)
    slab = jnp.sum(onehot * lab, axis=1, keepdims=True)

    lch = (sx - sx1) / 4.0 * sel_pos
    tch = (sy - sy1) / 4.0 * sel_pos
    rch = (sx2 - sx) / 4.0 * sel_pos
    bch = (sy2 - sy) / 4.0 * sel_pos
    regr_ref[0] = jnp.concatenate([lch, tch, rch, bch], axis=1)

    cls_iota = jax.lax.broadcasted_iota(
        jnp.int32, (TILE, NUM_CLASSES), 1).astype(jnp.float32)
    cls_ref[0] = (cls_iota == slab).astype(jnp.float32) * sel_pos

    any_ign = jnp.max(in_ign.astype(jnp.float32), axis=1, keepdims=True)
    clsm_ref[0] = jnp.maximum(sel_pos, 1.0 - any_ign)
    regrm_ref[0] = sel_pos

    part = jnp.sum(sel_pos).reshape(1, 1, 1)

    @pl.when(t == 0)
    def _init():
        npos_ref[...] = part

    @pl.when(t != 0)
    def _acc():
        npos_ref[...] += part


def kernel(gt_box_levels, gt_boxes, feature_shapes):
    del feature_shapes
    batch, nb = gt_box_levels.shape
    rows = jnp.concatenate(
        [jnp.transpose(gt_boxes, (0, 2, 1)),
         gt_box_levels[:, None, :].astype(jnp.float32)],
        axis=1,
    )
    meta = jnp.asarray(_META)

    cls_t, cls_m, num_pos, regr_t, regr_m = pl.pallas_call(
        _fsaf_tile,
        grid=(batch, NUM_TILES),
        in_specs=[
            pl.BlockSpec((1, 6, nb), lambda b, t: (b, 0, 0)),
            pl.BlockSpec((1, TILE, 8), lambda b, t: (t, 0, 0)),
        ],
        out_specs=[
            pl.BlockSpec((1, TILE, NUM_CLASSES), lambda b, t: (b, t, 0)),
            pl.BlockSpec((1, TILE, 1), lambda b, t: (b, t, 0)),
            pl.BlockSpec((1, 1, 1), lambda b, t: (b, 0, 0)),
            pl.BlockSpec((1, TILE, 4), lambda b, t: (b, t, 0)),
            pl.BlockSpec((1, TILE, 1), lambda b, t: (b, t, 0)),
        ],
        out_shape=[
            jax.ShapeDtypeStruct((batch, NUM_LOC, NUM_CLASSES), jnp.float32),
            jax.ShapeDtypeStruct((batch, NUM_LOC, 1), jnp.float32),
            jax.ShapeDtypeStruct((batch, 1, 1), jnp.float32),
            jax.ShapeDtypeStruct((batch, NUM_LOC, 4), jnp.float32),
            jax.ShapeDtypeStruct((batch, NUM_LOC, 1), jnp.float32),
        ],
    )(rows, meta)

    return (cls_t,
            cls_m[..., 0] != 0.0,
            num_pos[:, 0, 0],
            regr_t,
            regr_m[..., 0] != 0.0)
